# speedup vs baseline: 1.3089x; 1.3089x over previous
import jax
import jax.numpy as jnp
from jax.experimental import pallas as pl


def _add_kernel(emb_ref, pos_ref, out_ref):
    out_ref[...] = emb_ref[...] + pos_ref[...]


def kernel(embeddings, pos_table):
    B, S, D = embeddings.shape
    BS = 1024
    grid = (B, S // BS)
    return pl.pallas_call(
        _add_kernel,
        grid=grid,
        in_specs=[
            pl.BlockSpec((1, BS, D), lambda b, s: (b, s, 0)),
            pl.BlockSpec((BS, D), lambda b, s: (s, 0)),
        ],
        out_specs=pl.BlockSpec((1, BS, D), lambda b, s: (b, s, 0)),
        out_shape=jax.ShapeDtypeStruct((B, S, D), embeddings.dtype),
    )(embeddings, pos_table[:S])


# speedup vs baseline: 1.6594x; 1.2678x over previous
import jax
import jax.numpy as jnp
from jax.experimental import pallas as pl


def _add_kernel(emb_ref, pos_ref, out_ref):
    out_ref[...] = emb_ref[...] + pos_ref[...]


def kernel(embeddings, pos_table):
    B, S, D = embeddings.shape
    BS = 1024
    grid = (S // BS, B)
    return pl.pallas_call(
        _add_kernel,
        grid=grid,
        in_specs=[
            pl.BlockSpec((1, BS, D), lambda s, b: (b, s, 0)),
            pl.BlockSpec((BS, D), lambda s, b: (s, 0)),
        ],
        out_specs=pl.BlockSpec((1, BS, D), lambda s, b: (b, s, 0)),
        out_shape=jax.ShapeDtypeStruct((B, S, D), embeddings.dtype),
    )(embeddings, pos_table[:S])


# speedup vs baseline: 1.7379x; 1.0473x over previous
import jax
import jax.numpy as jnp
from jax.experimental import pallas as pl


def _add_kernel(emb_ref, pos_ref, out_ref):
    out_ref[...] = emb_ref[...] + pos_ref[...]


def kernel(embeddings, pos_table):
    B, S, D = embeddings.shape
    BS = 2048
    grid = (S // BS, B)
    return pl.pallas_call(
        _add_kernel,
        grid=grid,
        in_specs=[
            pl.BlockSpec((1, BS, D), lambda s, b: (b, s, 0)),
            pl.BlockSpec((BS, D), lambda s, b: (s, 0)),
        ],
        out_specs=pl.BlockSpec((1, BS, D), lambda s, b: (b, s, 0)),
        out_shape=jax.ShapeDtypeStruct((B, S, D), embeddings.dtype),
    )(embeddings, pos_table[:S])
